# baseline (speedup 1.0000x reference)
import functools

import jax
import jax.numpy as jnp
from jax import lax
from jax.experimental import pallas as pl
from jax.experimental.pallas import tpu as pltpu, tpu_sc as plsc

_ROWS = 8192
_DIM = 1024
_NC = 2
_NS = 16
_NW = _NC * _NS
_ROWS_PER_W = _ROWS // _NW

_mesh = plsc.VectorSubcoreMesh(core_axis_name="c", subcore_axis_name="s")


@functools.partial(
    pl.kernel,
    mesh=_mesh,
    out_type=jax.ShapeDtypeStruct((_ROWS, _DIM), jnp.float32),
)
def _copy_table(table_hbm, out_hbm):
    wid = lax.axis_index("s") * _NC + lax.axis_index("c")
    base = wid * _ROWS_PER_W
    pltpu.sync_copy(
        table_hbm.at[pl.ds(base, _ROWS_PER_W)],
        out_hbm.at[pl.ds(base, _ROWS_PER_W)],
    )


def kernel(x, table):
    return _copy_table(table)[None]


# speedup vs baseline: 24.1863x; 24.1863x over previous
import functools

import jax
import jax.numpy as jnp
from jax import lax
from jax.experimental import pallas as pl
from jax.experimental.pallas import tpu as pltpu, tpu_sc as plsc

_ROWS = 8192
_DIM = 1024
_NC = 2
_NS = 16
_NW = _NC * _NS
_ROWS_PER_W = _ROWS // _NW
_C = 32
_NCHUNK = _ROWS_PER_W // _C
_NBUF = 2

_mesh = plsc.VectorSubcoreMesh(core_axis_name="c", subcore_axis_name="s")


@functools.partial(
    pl.kernel,
    mesh=_mesh,
    out_type=jax.ShapeDtypeStruct((_ROWS, _DIM), jnp.float32),
    scratch_types=[
        pltpu.VMEM((_NBUF, _C, _DIM), jnp.float32),
        pltpu.SemaphoreType.DMA,
        pltpu.SemaphoreType.DMA,
        pltpu.SemaphoreType.DMA,
        pltpu.SemaphoreType.DMA,
    ],
)
def _copy_table(table_hbm, out_hbm, buf, si0, si1, so0, so1):
    wid = lax.axis_index("s") * _NC + lax.axis_index("c")
    base = wid * _ROWS_PER_W
    s_in = (si0, si1)
    s_out = (so0, so1)

    def cp_in(g, b):
        return pltpu.make_async_copy(
            table_hbm.at[pl.ds(base + g * _C, _C)], buf.at[b], s_in[b])

    def cp_out(g, b):
        return pltpu.make_async_copy(
            buf.at[b], out_hbm.at[pl.ds(base + g * _C, _C)], s_out[b])

    cp_in(0, 0).start()
    cp_in(1, 1).start()
    for g in range(_NCHUNK):
        b = g % _NBUF
        cp_in(g, b).wait()
        cp_out(g, b).start()
        if g + _NBUF < _NCHUNK:
            cp_out(g, b).wait()
            cp_in(g + _NBUF, b).start()
    for g in range(_NCHUNK - _NBUF, _NCHUNK):
        cp_out(g, g % _NBUF).wait()


def kernel(x, table):
    return _copy_table(table)[None]


# speedup vs baseline: 24.7344x; 1.0227x over previous
import functools

import jax
import jax.numpy as jnp
from jax import lax
from jax.experimental import pallas as pl
from jax.experimental.pallas import tpu as pltpu, tpu_sc as plsc

_ROWS = 8192
_DIM = 1024
_NC = 2
_NS = 16
_NW = _NC * _NS
_ROWS_PER_W = _ROWS // _NW
_C = 32
_NCHUNK = _ROWS_PER_W // _C
_NBUF = 3

_mesh = plsc.VectorSubcoreMesh(core_axis_name="c", subcore_axis_name="s")


@functools.partial(
    pl.kernel,
    mesh=_mesh,
    out_type=jax.ShapeDtypeStruct((_ROWS, _DIM), jnp.float32),
    scratch_types=[
        pltpu.VMEM((_NBUF, _C, _DIM), jnp.float32),
        pltpu.SemaphoreType.DMA,
        pltpu.SemaphoreType.DMA,
        pltpu.SemaphoreType.DMA,
        pltpu.SemaphoreType.DMA,
        pltpu.SemaphoreType.DMA,
        pltpu.SemaphoreType.DMA,
    ],
)
def _copy_table(table_hbm, out_hbm, buf, si0, si1, si2, so0, so1, so2):
    wid = lax.axis_index("s") * _NC + lax.axis_index("c")
    base = wid * _ROWS_PER_W
    s_in = (si0, si1, si2)
    s_out = (so0, so1, so2)

    def cp_in(g, b):
        return pltpu.make_async_copy(
            table_hbm.at[pl.ds(base + g * _C, _C)], buf.at[b], s_in[b])

    def cp_out(g, b):
        return pltpu.make_async_copy(
            buf.at[b], out_hbm.at[pl.ds(base + g * _C, _C)], s_out[b])

    for g in range(_NBUF):
        cp_in(g, g).start()
    for g in range(_NCHUNK):
        b = g % _NBUF
        cp_in(g, b).wait()
        cp_out(g, b).start()
        if g + _NBUF < _NCHUNK:
            cp_out(g, b).wait()
            cp_in(g + _NBUF, b).start()
    for g in range(_NCHUNK - _NBUF, _NCHUNK):
        cp_out(g, g % _NBUF).wait()


def kernel(x, table):
    return _copy_table(table)[None]
